# Initial kernel scaffold; baseline (speedup 1.0000x reference)
import jax
import jax.numpy as jnp
from jax.experimental import pallas as pl


def kernel(cat_mask, quaternion, scales, xy, z):
    raise NotImplementedError("write your pallas kernel here")


# speedup vs baseline: 9.5481x; 9.5481x over previous
import functools

import numpy as np
import jax
import jax.numpy as jnp
from jax import lax
from jax.experimental import pallas as pl
from jax.experimental.pallas import tpu as pltpu
from jax.experimental.pallas import tpu_sc as plsc

_NCLS = 9
_CM1 = 8
_INTR = np.array(
    [[572.4114, 0.0, 325.2611], [0.0, 573.57043, 242.04899], [0.0, 0.0, 1.0]],
    dtype=np.float32,
)
_KINV = np.linalg.inv(_INTR).astype(np.float32)

_B = 8
_HW = 224 * 224
_P = 448
_CPB = _HW // _P
_NW = 32
_CHUNKS = _B * _CPB
_CPT = _CHUNKS // _NW
_NV = _P // 16
_NSEG = _CM1 * _B
_NVAL = 11

_sc_mesh = plsc.VectorSubcoreMesh(core_axis_name="c", subcore_axis_name="s")


def _sc_body(cm_hbm, q_hbm, s_hbm, xy_hbm, z_hbm,
             gq_hbm, gs_hbm, gxy_hbm, gz_hbm, part_hbm,
             cm_v, q_v, s_v, xy_v, z_v, gq_v, gs_v, gxy_v, gz_v, acc_v):
    wid = lax.axis_index("s") * 2 + lax.axis_index("c")
    lanes = lax.iota(jnp.int32, 16)
    zf = jnp.zeros((16,), jnp.float32)

    def _zero(r, carry):
        for v in range(_NVAL):
            acc_v[r, v, :] = zf
        return carry

    lax.fori_loop(0, _NSEG, _zero, 0)

    def chunk_body(j, carry):
        g = wid * _CPT + j
        b = g // _CPB
        p0 = (g % _CPB) * _P
        pltpu.sync_copy(cm_hbm.at[b, pl.ds(p0, _P)], cm_v)
        pltpu.sync_copy(q_hbm.at[b, :, pl.ds(p0, _P)], q_v)
        pltpu.sync_copy(s_hbm.at[b, :, pl.ds(p0, _P)], s_v)
        pltpu.sync_copy(xy_hbm.at[b, :, pl.ds(p0, _P)], xy_v)
        pltpu.sync_copy(z_hbm.at[b, :, pl.ds(p0, _P)], z_v)

        def pix_body(i, c2):
            off = i * 16
            cls = cm_v[pl.ds(off, 16)]
            fg = cls > 0
            cm1 = jnp.clip(cls - 1, 0, _CM1 - 1)
            col = off + lanes
            seg = cm1 * _B + b
            fgf = jnp.where(fg, 1.0, 0.0).astype(jnp.float32)

            def _sel(src_v, ch, out_v, vbase):
                for k in range(ch):
                    row = cm1 * ch + k
                    gv = plsc.load_gather(src_v, [row, col])
                    gv = jnp.where(fg, gv, 0.0)
                    out_v[k, pl.ds(off, 16)] = gv
                    plsc.addupdate_scatter(
                        acc_v,
                        [seg, jnp.full((16,), vbase + k, jnp.int32), lanes],
                        gv,
                    )

            _sel(q_v, 4, gq_v, 0)
            _sel(s_v, 3, gs_v, 4)
            _sel(xy_v, 2, gxy_v, 7)
            gvz = plsc.load_gather(z_v, [cm1, col])
            gvz = jnp.where(fg, gvz, 0.0)
            gz_v[pl.ds(off, 16)] = gvz
            plsc.addupdate_scatter(
                acc_v, [seg, jnp.full((16,), 9, jnp.int32), lanes], gvz)
            plsc.addupdate_scatter(
                acc_v, [seg, jnp.full((16,), 10, jnp.int32), lanes], fgf)
            return c2

        lax.fori_loop(0, _NV, pix_body, 0)

        pltpu.sync_copy(gq_v, gq_hbm.at[b, :, pl.ds(p0, _P)])
        pltpu.sync_copy(gs_v, gs_hbm.at[b, :, pl.ds(p0, _P)])
        pltpu.sync_copy(gxy_v, gxy_hbm.at[b, :, pl.ds(p0, _P)])
        pltpu.sync_copy(gz_v, gz_hbm.at[b, pl.ds(p0, _P)])
        return carry

    lax.fori_loop(0, _CPT, chunk_body, 0)
    pltpu.sync_copy(acc_v, part_hbm.at[:, :, wid, :])


_sc_main = functools.partial(
    pl.kernel,
    out_type=(
        jax.ShapeDtypeStruct((_B, 4, _HW), jnp.float32),
        jax.ShapeDtypeStruct((_B, 3, _HW), jnp.float32),
        jax.ShapeDtypeStruct((_B, 2, _HW), jnp.float32),
        jax.ShapeDtypeStruct((_B, _HW), jnp.float32),
        jax.ShapeDtypeStruct((_NSEG, _NVAL, _NW, 16), jnp.float32),
    ),
    mesh=_sc_mesh,
    compiler_params=pltpu.CompilerParams(use_tc_tiling_on_sc=False,
                                         needs_layout_passes=False),
    scratch_types=[
        pltpu.VMEM((_P,), jnp.int32),
        pltpu.VMEM((4 * _CM1, _P), jnp.float32),
        pltpu.VMEM((3 * _CM1, _P), jnp.float32),
        pltpu.VMEM((2 * _CM1, _P), jnp.float32),
        pltpu.VMEM((1 * _CM1, _P), jnp.float32),
        pltpu.VMEM((4, _P), jnp.float32),
        pltpu.VMEM((3, _P), jnp.float32),
        pltpu.VMEM((2, _P), jnp.float32),
        pltpu.VMEM((_P,), jnp.float32),
        pltpu.VMEM((_NSEG, _NVAL, 16), jnp.float32),
    ],
)(_sc_body)


def _fin_body(p_ref, aq_ref, as_ref, axy_ref, az_ref, rt_ref, cnt_ref):
    w = _NW * 16
    cols = []
    for v in range(_NVAL):
        cols.append(jnp.sum(p_ref[:, v * w:(v + 1) * w], axis=1, keepdims=True))
    counts = cols[10]
    denom = jnp.maximum(counts, 1.0)
    q0, q1, q2, q3 = (cols[0] / denom, cols[1] / denom,
                      cols[2] / denom, cols[3] / denom)
    s0, s1, s2 = cols[4] / denom, cols[5] / denom, cols[6] / denom
    ax, ay = cols[7] / denom, cols[8] / denom
    azm = cols[9] / denom

    nrm = jnp.maximum(jnp.sqrt(q0 * q0 + q1 * q1 + q2 * q2 + q3 * q3), 1e-8)
    qw, qx, qy, qz = q0 / nrm, q1 / nrm, q2 / nrm, q3 / nrm
    r00 = 1.0 - 2.0 * (qy * qy + qz * qz)
    r01 = 2.0 * (qx * qy - qz * qw)
    r02 = 2.0 * (qx * qz + qy * qw)
    r10 = 2.0 * (qx * qy + qz * qw)
    r11 = 1.0 - 2.0 * (qx * qx + qz * qz)
    r12 = 2.0 * (qy * qz - qx * qw)
    r20 = 2.0 * (qx * qz - qy * qw)
    r21 = 2.0 * (qy * qz + qx * qw)
    r22 = 1.0 - 2.0 * (qx * qx + qy * qy)

    zval = jnp.exp(azm)
    ki = _KINV
    t0 = zval * (ki[0, 0] * ax + ki[0, 1] * ay + ki[0, 2])
    t1 = zval * (ki[1, 0] * ax + ki[1, 1] * ay + ki[1, 2])
    t2 = zval * (ki[2, 0] * ax + ki[2, 1] * ay + ki[2, 2])

    zc = jnp.zeros_like(q0)
    oc = jnp.ones_like(q0)
    rt_ref[:] = jnp.concatenate(
        [r00, r01, r02, t0,
         r10, r11, r12, t1,
         r20, r21, r22, t2,
         zc, zc, zc, oc], axis=1)
    aq_ref[:] = jnp.concatenate([q0, q1, q2, q3], axis=1)
    as_ref[:] = jnp.concatenate([s0, s1, s2], axis=1)
    axy_ref[:] = jnp.concatenate([ax, ay], axis=1)
    az_ref[:] = azm
    cnt_ref[:] = counts


_fin = pl.pallas_call(
    _fin_body,
    out_shape=(
        jax.ShapeDtypeStruct((_NSEG, 4), jnp.float32),
        jax.ShapeDtypeStruct((_NSEG, 3), jnp.float32),
        jax.ShapeDtypeStruct((_NSEG, 2), jnp.float32),
        jax.ShapeDtypeStruct((_NSEG, 1), jnp.float32),
        jax.ShapeDtypeStruct((_NSEG, 16), jnp.float32),
        jax.ShapeDtypeStruct((_NSEG, 1), jnp.float32),
    ),
)


@jax.jit
def kernel(cat_mask, quaternion, scales, xy, z):
    B, H, W = cat_mask.shape
    HW = H * W
    cm = cat_mask.reshape(B, HW).astype(jnp.int32)
    gq, gs, gxy, gz, part = _sc_main(
        cm,
        quaternion.reshape(B, 4 * _CM1, HW),
        scales.reshape(B, 3 * _CM1, HW),
        xy.reshape(B, 2 * _CM1, HW),
        z.reshape(B, _CM1, HW),
    )
    aq, ascl, axy_o, az, rt16, cnt = _fin(part.reshape(_NSEG, _NVAL * _NW * 16))
    RT = rt16.reshape(_NSEG, 4, 4)
    return (aq, ascl, axy_o, az, RT, cnt,
            gq.reshape(B, 4, H, W), gs.reshape(B, 3, H, W),
            gxy.reshape(B, 2, H, W), gz.reshape(B, H, W))


# speedup vs baseline: 13.4090x; 1.4044x over previous
import functools

import numpy as np
import jax
import jax.numpy as jnp
from jax import lax
from jax.experimental import pallas as pl
from jax.experimental.pallas import tpu as pltpu
from jax.experimental.pallas import tpu_sc as plsc

_NCLS = 9
_CM1 = 8
_INTR = np.array(
    [[572.4114, 0.0, 325.2611], [0.0, 573.57043, 242.04899], [0.0, 0.0, 1.0]],
    dtype=np.float32,
)
_KINV = np.linalg.inv(_INTR).astype(np.float32)

_B = 8
_HW = 224 * 224
_P = 448
_CPB = _HW // _P
_NW = 32
_CHUNKS = _B * _CPB
_CPT = _CHUNKS // _NW
_NV = _P // 16
_NSEG = _CM1 * _B
_NVAL = 11

_sc_mesh = plsc.VectorSubcoreMesh(core_axis_name="c", subcore_axis_name="s")


def _sc_body(cm_hbm, q_hbm, s_hbm, xy_hbm, z_hbm,
             gq_hbm, gs_hbm, gxy_hbm, gz_hbm, part_hbm,
             cm_b, q_b, s_b, xy_b, z_b, gq_b, gs_b, gxy_b, gz_b, acc_v,
             isem0, isem1, osem0, osem1):
    wid = lax.axis_index("s") * 2 + lax.axis_index("c")
    lanes = lax.iota(jnp.int32, 16)
    zf = jnp.zeros((16,), jnp.float32)
    isems = (isem0, isem1)
    osems = (osem0, osem1)

    def _zero(r, carry):
        for v in range(_NVAL):
            acc_v[r, v, :] = zf
        return carry

    lax.fori_loop(0, _NSEG, _zero, 0)

    def _bp(j):
        g = wid * _CPT + j
        b = g // _CPB
        p0 = (g % _CPB) * _P
        return b, p0

    def _in_descs(j, bi):
        b, p0 = _bp(j)
        return (
            (cm_hbm.at[b, pl.ds(p0, _P)], cm_b.at[bi]),
            (q_hbm.at[b, :, pl.ds(p0, _P)], q_b.at[bi]),
            (s_hbm.at[b, :, pl.ds(p0, _P)], s_b.at[bi]),
            (xy_hbm.at[b, :, pl.ds(p0, _P)], xy_b.at[bi]),
            (z_hbm.at[b, :, pl.ds(p0, _P)], z_b.at[bi]),
        )

    def _out_descs(j, bi):
        b, p0 = _bp(j)
        return (
            (gq_b.at[bi], gq_hbm.at[b, :, pl.ds(p0, _P)]),
            (gs_b.at[bi], gs_hbm.at[b, :, pl.ds(p0, _P)]),
            (gxy_b.at[bi], gxy_hbm.at[b, :, pl.ds(p0, _P)]),
            (gz_b.at[bi], gz_hbm.at[b, pl.ds(p0, _P)]),
        )

    def _issue(descs, sem):
        for src, dst in descs:
            pltpu.async_copy(src, dst, sem)

    def _drain(descs, sem):
        for src, dst in descs:
            pltpu.make_async_copy(src, dst, sem).wait()

    def _compute(j, bi):
        b, _ = _bp(j)
        q_v, s_v, xy_v, z_v = q_b.at[bi], s_b.at[bi], xy_b.at[bi], z_b.at[bi]
        gq_v, gs_v, gxy_v, gz_v = (gq_b.at[bi], gs_b.at[bi],
                                   gxy_b.at[bi], gz_b.at[bi])

        def pix_body(i, c2):
            off = i * 16
            cls = cm_b[bi, pl.ds(off, 16)]
            fg = cls > 0
            cm1 = jnp.clip(cls - 1, 0, _CM1 - 1)
            col = off + lanes
            seg = cm1 * _B + b
            fgf = jnp.where(fg, 1.0, 0.0).astype(jnp.float32)

            def _sel(src_v, ch, out_v, vbase):
                for k in range(ch):
                    row = cm1 * ch + k
                    gv = plsc.load_gather(src_v, [row, col])
                    gv = jnp.where(fg, gv, 0.0)
                    out_v[k, pl.ds(off, 16)] = gv
                    plsc.addupdate_scatter(
                        acc_v,
                        [seg, jnp.full((16,), vbase + k, jnp.int32), lanes],
                        gv,
                    )

            _sel(q_v, 4, gq_v, 0)
            _sel(s_v, 3, gs_v, 4)
            _sel(xy_v, 2, gxy_v, 7)
            gvz = plsc.load_gather(z_v, [cm1, col])
            gvz = jnp.where(fg, gvz, 0.0)
            gz_v[pl.ds(off, 16)] = gvz
            plsc.addupdate_scatter(
                acc_v, [seg, jnp.full((16,), 9, jnp.int32), lanes], gvz)
            plsc.addupdate_scatter(
                acc_v, [seg, jnp.full((16,), 10, jnp.int32), lanes], fgf)
            return c2

        lax.fori_loop(0, _NV, pix_body, 0)

    _issue(_in_descs(0, 0), isems[0])
    _issue(_in_descs(1, 1), isems[1])

    def pair_body(jj, carry):
        for bi in range(2):
            j = 2 * jj + bi
            _drain(_in_descs(j, bi), isems[bi])

            @pl.when(jj > 0)
            def _():
                _drain(_out_descs(j - 2, bi), osems[bi])

            _compute(j, bi)

            @pl.when(jj < (_CPT // 2) - 1)
            def _():
                _issue(_in_descs(j + 2, bi), isems[bi])

            _issue(_out_descs(j, bi), osems[bi])
        return carry

    lax.fori_loop(0, _CPT // 2, pair_body, 0)

    for bi in range(2):
        _drain(_out_descs(_CPT - 2 + bi, bi), osems[bi])
    pltpu.sync_copy(acc_v, part_hbm.at[:, :, wid, :])


_sc_main = functools.partial(
    pl.kernel,
    out_type=(
        jax.ShapeDtypeStruct((_B, 4, _HW), jnp.float32),
        jax.ShapeDtypeStruct((_B, 3, _HW), jnp.float32),
        jax.ShapeDtypeStruct((_B, 2, _HW), jnp.float32),
        jax.ShapeDtypeStruct((_B, _HW), jnp.float32),
        jax.ShapeDtypeStruct((_NSEG, _NVAL, _NW, 16), jnp.float32),
    ),
    mesh=_sc_mesh,
    compiler_params=pltpu.CompilerParams(use_tc_tiling_on_sc=False,
                                         needs_layout_passes=False),
    scratch_types=[
        pltpu.VMEM((2, _P), jnp.int32),
        pltpu.VMEM((2, 4 * _CM1, _P), jnp.float32),
        pltpu.VMEM((2, 3 * _CM1, _P), jnp.float32),
        pltpu.VMEM((2, 2 * _CM1, _P), jnp.float32),
        pltpu.VMEM((2, 1 * _CM1, _P), jnp.float32),
        pltpu.VMEM((2, 4, _P), jnp.float32),
        pltpu.VMEM((2, 3, _P), jnp.float32),
        pltpu.VMEM((2, 2, _P), jnp.float32),
        pltpu.VMEM((2, _P), jnp.float32),
        pltpu.VMEM((_NSEG, _NVAL, 16), jnp.float32),
        pltpu.SemaphoreType.DMA,
        pltpu.SemaphoreType.DMA,
        pltpu.SemaphoreType.DMA,
        pltpu.SemaphoreType.DMA,
    ],
)(_sc_body)


def _fin_body(p_ref, aq_ref, as_ref, axy_ref, az_ref, rt_ref, cnt_ref):
    w = _NW * 16
    cols = []
    for v in range(_NVAL):
        cols.append(jnp.sum(p_ref[:, v * w:(v + 1) * w], axis=1, keepdims=True))
    counts = cols[10]
    denom = jnp.maximum(counts, 1.0)
    q0, q1, q2, q3 = (cols[0] / denom, cols[1] / denom,
                      cols[2] / denom, cols[3] / denom)
    s0, s1, s2 = cols[4] / denom, cols[5] / denom, cols[6] / denom
    ax, ay = cols[7] / denom, cols[8] / denom
    azm = cols[9] / denom

    nrm = jnp.maximum(jnp.sqrt(q0 * q0 + q1 * q1 + q2 * q2 + q3 * q3), 1e-8)
    qw, qx, qy, qz = q0 / nrm, q1 / nrm, q2 / nrm, q3 / nrm
    r00 = 1.0 - 2.0 * (qy * qy + qz * qz)
    r01 = 2.0 * (qx * qy - qz * qw)
    r02 = 2.0 * (qx * qz + qy * qw)
    r10 = 2.0 * (qx * qy + qz * qw)
    r11 = 1.0 - 2.0 * (qx * qx + qz * qz)
    r12 = 2.0 * (qy * qz - qx * qw)
    r20 = 2.0 * (qx * qz - qy * qw)
    r21 = 2.0 * (qy * qz + qx * qw)
    r22 = 1.0 - 2.0 * (qx * qx + qy * qy)

    zval = jnp.exp(azm)
    ki = _KINV
    t0 = zval * (ki[0, 0] * ax + ki[0, 1] * ay + ki[0, 2])
    t1 = zval * (ki[1, 0] * ax + ki[1, 1] * ay + ki[1, 2])
    t2 = zval * (ki[2, 0] * ax + ki[2, 1] * ay + ki[2, 2])

    zc = jnp.zeros_like(q0)
    oc = jnp.ones_like(q0)
    rt_ref[:] = jnp.concatenate(
        [r00, r01, r02, t0,
         r10, r11, r12, t1,
         r20, r21, r22, t2,
         zc, zc, zc, oc], axis=1)
    aq_ref[:] = jnp.concatenate([q0, q1, q2, q3], axis=1)
    as_ref[:] = jnp.concatenate([s0, s1, s2], axis=1)
    axy_ref[:] = jnp.concatenate([ax, ay], axis=1)
    az_ref[:] = azm
    cnt_ref[:] = counts


_fin = pl.pallas_call(
    _fin_body,
    out_shape=(
        jax.ShapeDtypeStruct((_NSEG, 4), jnp.float32),
        jax.ShapeDtypeStruct((_NSEG, 3), jnp.float32),
        jax.ShapeDtypeStruct((_NSEG, 2), jnp.float32),
        jax.ShapeDtypeStruct((_NSEG, 1), jnp.float32),
        jax.ShapeDtypeStruct((_NSEG, 16), jnp.float32),
        jax.ShapeDtypeStruct((_NSEG, 1), jnp.float32),
    ),
)


@jax.jit
def kernel(cat_mask, quaternion, scales, xy, z):
    B, H, W = cat_mask.shape
    HW = H * W
    cm = cat_mask.reshape(B, HW).astype(jnp.int32)
    gq, gs, gxy, gz, part = _sc_main(
        cm,
        quaternion.reshape(B, 4 * _CM1, HW),
        scales.reshape(B, 3 * _CM1, HW),
        xy.reshape(B, 2 * _CM1, HW),
        z.reshape(B, _CM1, HW),
    )
    aq, ascl, axy_o, az, rt16, cnt = _fin(part.reshape(_NSEG, _NVAL * _NW * 16))
    RT = rt16.reshape(_NSEG, 4, 4)
    return (aq, ascl, axy_o, az, RT, cnt,
            gq.reshape(B, 4, H, W), gs.reshape(B, 3, H, W),
            gxy.reshape(B, 2, H, W), gz.reshape(B, H, W))


# speedup vs baseline: 18.7113x; 1.3954x over previous
import functools

import numpy as np
import jax
import jax.numpy as jnp
from jax import lax
from jax.experimental import pallas as pl
from jax.experimental.pallas import tpu as pltpu
from jax.experimental.pallas import tpu_sc as plsc

_NCLS = 9
_CM1 = 8
_INTR = np.array(
    [[572.4114, 0.0, 325.2611], [0.0, 573.57043, 242.04899], [0.0, 0.0, 1.0]],
    dtype=np.float32,
)
_KINV = np.linalg.inv(_INTR).astype(np.float32)

_B = 8
_H = 224
_W = 224
_NW = 32
_HB = _H // 8
_TPT = (_B * _HB) // _NW
_NSEG = _CM1 * _B
_NVAL = 11
_ACCW = _NSEG * 16
_ACCN = _NVAL * _ACCW
_VPR = _W // 16
_NVREG = 8 * _VPR

_GCH = (4, 3, 2, 1)
_GVB = (0, 4, 7, 9)

_sc_mesh = plsc.VectorSubcoreMesh(core_axis_name="c", subcore_axis_name="s")


def _sc_body(cm_hbm, q_hbm, s_hbm, xy_hbm, z_hbm,
             gq_hbm, gs_hbm, gxy_hbm, gz_hbm, part_hbm,
             cm_v, a_v, b_v, gq_v, gs_v, gxy_v, gz_v, acc_v,
             m_s, a_s, b_s, oq_s, os_s, oxy_s, oz_s):
    wid = lax.axis_index("s") * 2 + lax.axis_index("c")
    lanes = lax.iota(jnp.int32, 16)
    zf = jnp.zeros((16,), jnp.float32)
    in_hbm = (q_hbm, s_hbm, xy_hbm, z_hbm)
    out_v = (gq_v, gs_v, gxy_v, gz_v)
    out_hbm = (gq_hbm, gs_hbm, gxy_hbm, gz_hbm)
    out_sems = (oq_s, os_s, oxy_s, oz_s)

    def _zero(i, carry):
        acc_v[pl.ds(i * 16, 16)] = zf
        return carry

    lax.fori_loop(0, _ACCN // 16, _zero, 0)

    def _coords(jt):
        idx = wid * _TPT + jt
        b = idx // _HB
        h0 = pl.multiple_of((idx % _HB) * 8, 8)
        return b, h0

    def _in_desc(sp):
        jt, pi = sp // 8, sp % 8
        g, hf = pi // 2, pi % 2
        ch = _GCH[g]
        b, h0 = _coords(jt)
        nch = 4 * ch
        src = in_hbm[g].at[b, pl.ds(hf * nch, nch), pl.ds(h0, 8), :]
        buf = a_v if pi % 2 == 0 else b_v
        return src, buf.at[pl.ds(0, nch)], (a_s if pi % 2 == 0 else b_s)

    def _m_desc(jt, pb):
        b, h0 = _coords(jt)
        return cm_hbm.at[b, pl.ds(h0, 8), :], cm_v.at[pb]

    def _out_desc(jt, g):
        b, h0 = _coords(jt)
        if g == 3:
            return gz_v, gz_hbm.at[b, pl.ds(h0, 8), :]
        return out_v[g], out_hbm[g].at[b, :, pl.ds(h0, 8), :]

    def _compute(sp, pb):
        jt, pi = sp // 8, sp % 8
        g, hf = pi // 2, pi % 2
        ch = _GCH[g]
        vb = _GVB[g]
        b, _ = _coords(jt)
        buf = a_v if pi % 2 == 0 else b_v
        ov = out_v[g]

        def body(i, carry):
            hh = i // _VPR
            lo = (i % _VPR) * 16
            cls = cm_v[pb, hh, pl.ds(lo, 16)]
            fg = cls > 0
            cm1 = jnp.clip(cls - 1, 0, _CM1 - 1)
            lcm1 = jnp.clip(cm1 - hf * 4, 0, 3)
            sel = jnp.logical_and(fg, (cm1 // 4) == hf)
            seg16 = (cm1 * _B + b) * 16 + lanes
            hhv = jnp.full((16,), hh, jnp.int32)
            ww = lo + lanes
            for k in range(ch):
                row = lcm1 * ch + k
                gv = plsc.load_gather(buf, [row, hhv, ww])
                gv = jnp.where(sel, gv, 0.0)
                if g == 3:
                    if hf == 0:
                        gz_v[hh, pl.ds(lo, 16)] = gv
                    else:
                        gz_v[hh, pl.ds(lo, 16)] = gz_v[hh, pl.ds(lo, 16)] + gv
                else:
                    if hf == 0:
                        ov[k, hh, pl.ds(lo, 16)] = gv
                    else:
                        ov[k, hh, pl.ds(lo, 16)] = ov[k, hh, pl.ds(lo, 16)] + gv
                plsc.addupdate_scatter(acc_v, [(vb + k) * _ACCW + seg16], gv)
            if g == 0 and hf == 0:
                fgf = jnp.where(fg, 1.0, 0.0).astype(jnp.float32)
                plsc.addupdate_scatter(acc_v, [10 * _ACCW + seg16], fgf)
            return carry

        lax.fori_loop(0, _NVREG, body, 0)

    NSP = 8 * _TPT

    def _issue(d):
        pltpu.async_copy(d[0], d[1], d[2])

    _issue(_m_desc(0, 0) + (m_s,))
    _issue(_in_desc(0))
    _issue(_in_desc(1))

    for sp in range(NSP):
        jt, pi = sp // 8, sp % 8
        pb = jt % 2
        if pi == 0:
            src, dst = _m_desc(jt, pb)
            pltpu.make_async_copy(src, dst, m_s).wait()
            if jt + 1 < _TPT:
                _issue(_m_desc(jt + 1, 1 - pb) + (m_s,))
        src, dst, sem = _in_desc(sp)
        pltpu.make_async_copy(src, dst, sem).wait()
        if pi % 2 == 0 and jt > 0:
            g = pi // 2
            osrc, odst = _out_desc(jt - 1, g)
            pltpu.make_async_copy(osrc, odst, out_sems[g]).wait()
        _compute(sp, pb)
        if sp + 2 < NSP:
            _issue(_in_desc(sp + 2))
        if pi % 2 == 1:
            g = pi // 2
            osrc, odst = _out_desc(jt, g)
            pltpu.async_copy(osrc, odst, out_sems[g])

    for g in range(4):
        osrc, odst = _out_desc(_TPT - 1, g)
        pltpu.make_async_copy(osrc, odst, out_sems[g]).wait()
    off = pl.multiple_of(wid * _ACCN, 128)
    pltpu.sync_copy(acc_v, part_hbm.at[pl.ds(off, _ACCN)])


_sc_main = functools.partial(
    pl.kernel,
    out_type=(
        jax.ShapeDtypeStruct((_B, 4, _H, _W), jnp.float32),
        jax.ShapeDtypeStruct((_B, 3, _H, _W), jnp.float32),
        jax.ShapeDtypeStruct((_B, 2, _H, _W), jnp.float32),
        jax.ShapeDtypeStruct((_B, _H, _W), jnp.float32),
        jax.ShapeDtypeStruct((_NW * _ACCN,), jnp.float32),
    ),
    mesh=_sc_mesh,
    compiler_params=pltpu.CompilerParams(needs_layout_passes=False),
    scratch_types=[
        pltpu.VMEM((2, 8, _W), jnp.int32),
        pltpu.VMEM((16, 8, _W), jnp.float32),
        pltpu.VMEM((16, 8, _W), jnp.float32),
        pltpu.VMEM((4, 8, _W), jnp.float32),
        pltpu.VMEM((3, 8, _W), jnp.float32),
        pltpu.VMEM((2, 8, _W), jnp.float32),
        pltpu.VMEM((8, _W), jnp.float32),
        pltpu.VMEM((_ACCN,), jnp.float32),
        pltpu.SemaphoreType.DMA,
        pltpu.SemaphoreType.DMA,
        pltpu.SemaphoreType.DMA,
        pltpu.SemaphoreType.DMA,
        pltpu.SemaphoreType.DMA,
        pltpu.SemaphoreType.DMA,
        pltpu.SemaphoreType.DMA,
    ],
)(_sc_body)


def _fin_body(p_ref, aq_ref, as_ref, axy_ref, az_ref, rt_ref, cnt_ref):
    w = _NW * 16
    cols = []
    for v in range(_NVAL):
        cols.append(jnp.sum(p_ref[:, v * w:(v + 1) * w], axis=1, keepdims=True))
    counts = cols[10]
    denom = jnp.maximum(counts, 1.0)
    q0, q1, q2, q3 = (cols[0] / denom, cols[1] / denom,
                      cols[2] / denom, cols[3] / denom)
    s0, s1, s2 = cols[4] / denom, cols[5] / denom, cols[6] / denom
    ax, ay = cols[7] / denom, cols[8] / denom
    azm = cols[9] / denom

    nrm = jnp.maximum(jnp.sqrt(q0 * q0 + q1 * q1 + q2 * q2 + q3 * q3), 1e-8)
    qw, qx, qy, qz = q0 / nrm, q1 / nrm, q2 / nrm, q3 / nrm
    r00 = 1.0 - 2.0 * (qy * qy + qz * qz)
    r01 = 2.0 * (qx * qy - qz * qw)
    r02 = 2.0 * (qx * qz + qy * qw)
    r10 = 2.0 * (qx * qy + qz * qw)
    r11 = 1.0 - 2.0 * (qx * qx + qz * qz)
    r12 = 2.0 * (qy * qz - qx * qw)
    r20 = 2.0 * (qx * qz - qy * qw)
    r21 = 2.0 * (qy * qz + qx * qw)
    r22 = 1.0 - 2.0 * (qx * qx + qy * qy)

    zval = jnp.exp(azm)
    ki = _KINV
    t0 = zval * (ki[0, 0] * ax + ki[0, 1] * ay + ki[0, 2])
    t1 = zval * (ki[1, 0] * ax + ki[1, 1] * ay + ki[1, 2])
    t2 = zval * (ki[2, 0] * ax + ki[2, 1] * ay + ki[2, 2])

    zc = jnp.zeros_like(q0)
    oc = jnp.ones_like(q0)
    rt_ref[:] = jnp.concatenate(
        [r00, r01, r02, t0,
         r10, r11, r12, t1,
         r20, r21, r22, t2,
         zc, zc, zc, oc], axis=1)
    aq_ref[:] = jnp.concatenate([q0, q1, q2, q3], axis=1)
    as_ref[:] = jnp.concatenate([s0, s1, s2], axis=1)
    axy_ref[:] = jnp.concatenate([ax, ay], axis=1)
    az_ref[:] = azm
    cnt_ref[:] = counts


_fin = pl.pallas_call(
    _fin_body,
    out_shape=(
        jax.ShapeDtypeStruct((_NSEG, 4), jnp.float32),
        jax.ShapeDtypeStruct((_NSEG, 3), jnp.float32),
        jax.ShapeDtypeStruct((_NSEG, 2), jnp.float32),
        jax.ShapeDtypeStruct((_NSEG, 1), jnp.float32),
        jax.ShapeDtypeStruct((_NSEG, 16), jnp.float32),
        jax.ShapeDtypeStruct((_NSEG, 1), jnp.float32),
    ),
)


@jax.jit
def kernel(cat_mask, quaternion, scales, xy, z):
    gq, gs, gxy, gz, part = _sc_main(
        cat_mask.astype(jnp.int32), quaternion, scales, xy, z)
    p4 = part.reshape(_NW, _NVAL, _NSEG, 16).transpose(2, 1, 0, 3)
    aq, ascl, axy_o, az, rt16, cnt = _fin(p4.reshape(_NSEG, _NVAL * _NW * 16))
    RT = rt16.reshape(_NSEG, 4, 4)
    return aq, ascl, axy_o, az, RT, cnt, gq, gs, gxy, gz


# speedup vs baseline: 23.1320x; 1.2363x over previous
import functools

import numpy as np
import jax
import jax.numpy as jnp
from jax import lax
from jax.experimental import pallas as pl
from jax.experimental.pallas import tpu as pltpu
from jax.experimental.pallas import tpu_sc as plsc

_NCLS = 9
_CM1 = 8
_INTR = np.array(
    [[572.4114, 0.0, 325.2611], [0.0, 573.57043, 242.04899], [0.0, 0.0, 1.0]],
    dtype=np.float32,
)
_KINV = np.linalg.inv(_INTR).astype(np.float32)

_B = 8
_H = 224
_W = 224
_NW = 32
_HB = _H // 8
_TPT = (_B * _HB) // _NW
_NSEG = _CM1 * _B
_NVAL = 11
_ACCW = _NSEG * 16
_ACCN = _NVAL * _ACCW
_VPR = _W // 16
_NVREG = 8 * _VPR

_GCH = (4, 3, 2, 1)
_GVB = (0, 4, 7, 9)

_sc_mesh = plsc.VectorSubcoreMesh(core_axis_name="c", subcore_axis_name="s")


def _sc_body(cm_hbm, q_hbm, s_hbm, xy_hbm, z_hbm,
             gq_hbm, gs_hbm, gxy_hbm, gz_hbm, part_hbm,
             cm_v, a_v, b_v, o_v, acc_v,
             m_s, a_s, b_s, oq_s, os_s, oxy_s, oz_s):
    wid = lax.axis_index("s") * 2 + lax.axis_index("c")
    lanes = lax.iota(jnp.int32, 16)
    zf = jnp.zeros((16,), jnp.float32)
    out_sems = (oq_s, os_s, oxy_s, oz_s)

    def _zero(i, carry):
        acc_v[pl.ds(i * 16, 16)] = zf
        return carry

    lax.fori_loop(0, _ACCN // 16, _zero, 0)

    def _coords(jt):
        idx = wid * _TPT + jt
        b = idx // _HB
        h0 = pl.multiple_of((idx % _HB) * 8, 8)
        return b, h0

    def _in_desc(jt, st):
        b, h0 = _coords(jt)
        if st == 0:
            return (q_hbm.at[b, pl.ds(0, 16), pl.ds(h0, 8), :],
                    a_v.at[pl.ds(0, 16)], a_s)
        if st == 1:
            return (q_hbm.at[b, pl.ds(16, 16), pl.ds(h0, 8), :],
                    b_v.at[pl.ds(0, 16)], b_s)
        if st == 2:
            return (s_hbm.at[b, :, pl.ds(h0, 8), :], b_v, b_s)
        if st == 3:
            return (xy_hbm.at[b, :, pl.ds(h0, 8), :],
                    a_v.at[pl.ds(0, 16)], a_s)
        return (z_hbm.at[b, :, pl.ds(h0, 8), :], b_v.at[pl.ds(0, 8)], b_s)

    def _m_desc(jt, pb):
        b, h0 = _coords(jt)
        return cm_hbm.at[b, pl.ds(h0, 8), :], cm_v.at[pb], m_s

    def _out_desc(jt, g):
        b, h0 = _coords(jt)
        if g == 0:
            return o_v.at[pl.ds(0, 4)], gq_hbm.at[b, :, pl.ds(h0, 8), :]
        if g == 1:
            return o_v.at[pl.ds(4, 3)], gs_hbm.at[b, :, pl.ds(h0, 8), :]
        if g == 2:
            return o_v.at[pl.ds(7, 2)], gxy_hbm.at[b, :, pl.ds(h0, 8), :]
        return o_v.at[9], gz_hbm.at[b, pl.ds(h0, 8), :]

    def _issue(d):
        pltpu.async_copy(d[0], d[1], d[2])

    def _wait(d):
        pltpu.make_async_copy(d[0], d[1], d[2]).wait()

    def _wait_out(jt, g):
        osrc, odst = _out_desc(jt, g)
        pltpu.make_async_copy(osrc, odst, out_sems[g]).wait()

    def _issue_out(jt, g):
        osrc, odst = _out_desc(jt, g)
        pltpu.async_copy(osrc, odst, out_sems[g])

    def _pass(jt, pb, g):
        ch = _GCH[g]
        vb = _GVB[g]
        orow = (0, 4, 7, 9)[g]
        b, _ = _coords(jt)

        def body(i, carry):
            hh = i // _VPR
            lo = (i % _VPR) * 16
            cls = cm_v[pb, hh, pl.ds(lo, 16)]
            fg = cls > 0
            cm1 = jnp.clip(cls - 1, 0, _CM1 - 1)
            seg16 = (cm1 * _B + b) * 16 + lanes
            hhv = jnp.full((16,), hh, jnp.int32)
            ww = lo + lanes
            if g == 0:
                lrow = jnp.clip(cm1, 0, 3) * 4
                hrow = jnp.clip(cm1 - 4, 0, 3) * 4
                hi = cm1 >= 4
            for k in range(ch):
                if g == 0:
                    g_lo = plsc.load_gather(a_v, [lrow + k, hhv, ww])
                    g_hi = plsc.load_gather(b_v, [hrow + k, hhv, ww])
                    gv = jnp.where(hi, g_hi, g_lo)
                elif g == 1:
                    gv = plsc.load_gather(b_v, [cm1 * 3 + k, hhv, ww])
                elif g == 2:
                    gv = plsc.load_gather(a_v, [cm1 * 2 + k, hhv, ww])
                else:
                    gv = plsc.load_gather(b_v, [cm1, hhv, ww])
                gv = jnp.where(fg, gv, 0.0)
                o_v[orow + k, hh, pl.ds(lo, 16)] = gv
                plsc.addupdate_scatter(acc_v, [(vb + k) * _ACCW + seg16], gv)
            if g == 0:
                fgf = jnp.where(fg, 1.0, 0.0).astype(jnp.float32)
                plsc.addupdate_scatter(acc_v, [10 * _ACCW + seg16], fgf)
            return carry

        lax.fori_loop(0, _NVREG, body, 0)

    _issue(_m_desc(0, 0))
    _issue(_in_desc(0, 0))
    _issue(_in_desc(0, 1))

    for jt in range(_TPT):
        pb = jt % 2
        _wait(_m_desc(jt, pb))
        if jt + 1 < _TPT:
            _issue(_m_desc(jt + 1, 1 - pb))
        _wait(_in_desc(jt, 0))
        _wait(_in_desc(jt, 1))
        if jt > 0:
            _wait_out(jt - 1, 0)
        _pass(jt, pb, 0)
        _issue(_in_desc(jt, 2))
        _issue(_in_desc(jt, 3))
        _issue_out(jt, 0)
        _wait(_in_desc(jt, 2))
        if jt > 0:
            _wait_out(jt - 1, 1)
        _pass(jt, pb, 1)
        _issue(_in_desc(jt, 4))
        _issue_out(jt, 1)
        _wait(_in_desc(jt, 3))
        if jt > 0:
            _wait_out(jt - 1, 2)
        _pass(jt, pb, 2)
        if jt + 1 < _TPT:
            _issue(_in_desc(jt + 1, 0))
        _issue_out(jt, 2)
        _wait(_in_desc(jt, 4))
        if jt > 0:
            _wait_out(jt - 1, 3)
        _pass(jt, pb, 3)
        if jt + 1 < _TPT:
            _issue(_in_desc(jt + 1, 1))
        _issue_out(jt, 3)

    for g in range(4):
        _wait_out(_TPT - 1, g)
    off = pl.multiple_of(wid * _ACCN, 128)
    pltpu.sync_copy(acc_v, part_hbm.at[pl.ds(off, _ACCN)])


_sc_main = functools.partial(
    pl.kernel,
    out_type=(
        jax.ShapeDtypeStruct((_B, 4, _H, _W), jnp.float32),
        jax.ShapeDtypeStruct((_B, 3, _H, _W), jnp.float32),
        jax.ShapeDtypeStruct((_B, 2, _H, _W), jnp.float32),
        jax.ShapeDtypeStruct((_B, _H, _W), jnp.float32),
        jax.ShapeDtypeStruct((_NW * _ACCN,), jnp.float32),
    ),
    mesh=_sc_mesh,
    compiler_params=pltpu.CompilerParams(needs_layout_passes=False),
    scratch_types=[
        pltpu.VMEM((2, 8, _W), jnp.int32),
        pltpu.VMEM((16, 8, _W), jnp.float32),
        pltpu.VMEM((24, 8, _W), jnp.float32),
        pltpu.VMEM((10, 8, _W), jnp.float32),
        pltpu.VMEM((_ACCN,), jnp.float32),
        pltpu.SemaphoreType.DMA,
        pltpu.SemaphoreType.DMA,
        pltpu.SemaphoreType.DMA,
        pltpu.SemaphoreType.DMA,
        pltpu.SemaphoreType.DMA,
        pltpu.SemaphoreType.DMA,
        pltpu.SemaphoreType.DMA,
    ],
)(_sc_body)


def _fin_body(p_ref, aq_ref, as_ref, axy_ref, az_ref, rt_ref, cnt_ref):
    w = _NW * 16
    cols = []
    for v in range(_NVAL):
        cols.append(jnp.sum(p_ref[:, v * w:(v + 1) * w], axis=1, keepdims=True))
    counts = cols[10]
    denom = jnp.maximum(counts, 1.0)
    q0, q1, q2, q3 = (cols[0] / denom, cols[1] / denom,
                      cols[2] / denom, cols[3] / denom)
    s0, s1, s2 = cols[4] / denom, cols[5] / denom, cols[6] / denom
    ax, ay = cols[7] / denom, cols[8] / denom
    azm = cols[9] / denom

    nrm = jnp.maximum(jnp.sqrt(q0 * q0 + q1 * q1 + q2 * q2 + q3 * q3), 1e-8)
    qw, qx, qy, qz = q0 / nrm, q1 / nrm, q2 / nrm, q3 / nrm
    r00 = 1.0 - 2.0 * (qy * qy + qz * qz)
    r01 = 2.0 * (qx * qy - qz * qw)
    r02 = 2.0 * (qx * qz + qy * qw)
    r10 = 2.0 * (qx * qy + qz * qw)
    r11 = 1.0 - 2.0 * (qx * qx + qz * qz)
    r12 = 2.0 * (qy * qz - qx * qw)
    r20 = 2.0 * (qx * qz - qy * qw)
    r21 = 2.0 * (qy * qz + qx * qw)
    r22 = 1.0 - 2.0 * (qx * qx + qy * qy)

    zval = jnp.exp(azm)
    ki = _KINV
    t0 = zval * (ki[0, 0] * ax + ki[0, 1] * ay + ki[0, 2])
    t1 = zval * (ki[1, 0] * ax + ki[1, 1] * ay + ki[1, 2])
    t2 = zval * (ki[2, 0] * ax + ki[2, 1] * ay + ki[2, 2])

    zc = jnp.zeros_like(q0)
    oc = jnp.ones_like(q0)
    rt_ref[:] = jnp.concatenate(
        [r00, r01, r02, t0,
         r10, r11, r12, t1,
         r20, r21, r22, t2,
         zc, zc, zc, oc], axis=1)
    aq_ref[:] = jnp.concatenate([q0, q1, q2, q3], axis=1)
    as_ref[:] = jnp.concatenate([s0, s1, s2], axis=1)
    axy_ref[:] = jnp.concatenate([ax, ay], axis=1)
    az_ref[:] = azm
    cnt_ref[:] = counts


_fin = pl.pallas_call(
    _fin_body,
    out_shape=(
        jax.ShapeDtypeStruct((_NSEG, 4), jnp.float32),
        jax.ShapeDtypeStruct((_NSEG, 3), jnp.float32),
        jax.ShapeDtypeStruct((_NSEG, 2), jnp.float32),
        jax.ShapeDtypeStruct((_NSEG, 1), jnp.float32),
        jax.ShapeDtypeStruct((_NSEG, 16), jnp.float32),
        jax.ShapeDtypeStruct((_NSEG, 1), jnp.float32),
    ),
)


@jax.jit
def kernel(cat_mask, quaternion, scales, xy, z):
    gq, gs, gxy, gz, part = _sc_main(
        cat_mask.astype(jnp.int32), quaternion, scales, xy, z)
    p4 = part.reshape(_NW, _NVAL, _NSEG, 16).transpose(2, 1, 0, 3)
    aq, ascl, axy_o, az, rt16, cnt = _fin(p4.reshape(_NSEG, _NVAL * _NW * 16))
    RT = rt16.reshape(_NSEG, 4, 4)
    return aq, ascl, axy_o, az, RT, cnt, gq, gs, gxy, gz


# speedup vs baseline: 24.1412x; 1.0436x over previous
import functools

import numpy as np
import jax
import jax.numpy as jnp
from jax import lax
from jax.experimental import pallas as pl
from jax.experimental.pallas import tpu as pltpu
from jax.experimental.pallas import tpu_sc as plsc

_NCLS = 9
_CM1 = 8
_INTR = np.array(
    [[572.4114, 0.0, 325.2611], [0.0, 573.57043, 242.04899], [0.0, 0.0, 1.0]],
    dtype=np.float32,
)
_KINV = np.linalg.inv(_INTR).astype(np.float32)

_B = 8
_H = 224
_W = 224
_NW = 32
_HB = _H // 8
_TPT = (_B * _HB) // _NW
_NSEG = _CM1 * _B
_NVAL = 11
_ACCW = _NSEG * 16
_ACCN = _NVAL * _ACCW
_VPR = _W // 16
_NVREG = 8 * _VPR

_GCH = (4, 3, 2, 1)
_GVB = (0, 4, 7, 9)

_sc_mesh = plsc.VectorSubcoreMesh(core_axis_name="c", subcore_axis_name="s")


def _sc_body(cm_hbm, q_hbm, s_hbm, xy_hbm, z_hbm,
             gq_hbm, gs_hbm, gxy_hbm, gz_hbm, part_hbm,
             cm_v, a_v, b_v, o_v, acc_v,
             m_s, a_s, b_s, z_s, oq_s, os_s, oxy_s, oz_s):
    wid = lax.axis_index("s") * 2 + lax.axis_index("c")
    lanes = lax.iota(jnp.int32, 16)
    zf = jnp.zeros((16,), jnp.float32)
    out_sems = (oq_s, os_s, oxy_s, oz_s)

    def _zero(i, carry):
        acc_v[pl.ds(i * 16, 16)] = zf
        return carry

    lax.fori_loop(0, _ACCN // 16, _zero, 0)

    def _coords(jt):
        idx = wid * _TPT + jt
        b = idx // _HB
        h0 = pl.multiple_of((idx % _HB) * 8, 8)
        return b, h0

    def _in_desc(jt, st):
        b, h0 = _coords(jt)
        if st == 0:
            return (q_hbm.at[b, pl.ds(0, 16), pl.ds(h0, 8), :],
                    a_v.at[pl.ds(0, 16)], a_s)
        if st == 1:
            return (q_hbm.at[b, pl.ds(16, 16), pl.ds(h0, 8), :],
                    b_v.at[pl.ds(0, 16)], b_s)
        if st == 2:
            return (s_hbm.at[b, :, pl.ds(h0, 8), :], b_v, b_s)
        if st == 3:
            return (xy_hbm.at[b, :, pl.ds(h0, 8), :],
                    a_v.at[pl.ds(0, 16)], a_s)
        return (z_hbm.at[b, :, pl.ds(h0, 8), :], a_v.at[pl.ds(16, 8)], z_s)

    def _m_desc(jt, pb):
        b, h0 = _coords(jt)
        return cm_hbm.at[b, pl.ds(h0, 8), :], cm_v.at[pb], m_s

    def _out_desc(jt, g):
        b, h0 = _coords(jt)
        if g == 0:
            return o_v.at[pl.ds(0, 4)], gq_hbm.at[b, :, pl.ds(h0, 8), :]
        if g == 1:
            return o_v.at[pl.ds(0, 3)], gs_hbm.at[b, :, pl.ds(h0, 8), :]
        if g == 2:
            return o_v.at[pl.ds(0, 2)], gxy_hbm.at[b, :, pl.ds(h0, 8), :]
        return o_v.at[2], gz_hbm.at[b, pl.ds(h0, 8), :]

    def _issue(d):
        pltpu.async_copy(d[0], d[1], d[2])

    def _wait(d):
        pltpu.make_async_copy(d[0], d[1], d[2]).wait()

    def _wait_out(jt, g):
        osrc, odst = _out_desc(jt, g)
        pltpu.make_async_copy(osrc, odst, out_sems[g]).wait()

    def _issue_out(jt, g):
        osrc, odst = _out_desc(jt, g)
        pltpu.async_copy(osrc, odst, out_sems[g])

    def _pass(jt, pb, g):
        ch = _GCH[g]
        vb = _GVB[g]
        orow = (0, 0, 0, 2)[g]
        b, _ = _coords(jt)

        def body(i, carry):
            hh = i // _VPR
            lo = (i % _VPR) * 16
            cls = cm_v[pb, hh, pl.ds(lo, 16)]
            fg = cls > 0
            cm1 = jnp.clip(cls - 1, 0, _CM1 - 1)
            seg16 = (cm1 * _B + b) * 16 + lanes
            hhv = jnp.full((16,), hh, jnp.int32)
            ww = lo + lanes
            if g == 0:
                lrow = jnp.clip(cm1, 0, 3) * 4
                hrow = jnp.clip(cm1 - 4, 0, 3) * 4
                hi = cm1 >= 4
            for k in range(ch):
                if g == 0:
                    g_lo = plsc.load_gather(a_v, [lrow + k, hhv, ww])
                    g_hi = plsc.load_gather(b_v, [hrow + k, hhv, ww])
                    gv = jnp.where(hi, g_hi, g_lo)
                elif g == 1:
                    gv = plsc.load_gather(b_v, [cm1 * 3 + k, hhv, ww])
                elif g == 2:
                    gv = plsc.load_gather(a_v, [cm1 * 2 + k, hhv, ww])
                else:
                    gv = plsc.load_gather(a_v, [16 + cm1, hhv, ww])
                gv = jnp.where(fg, gv, 0.0)
                o_v[orow + k, hh, pl.ds(lo, 16)] = gv
                plsc.addupdate_scatter(acc_v, [(vb + k) * _ACCW + seg16], gv)
            if g == 0:
                fgf = jnp.where(fg, 1.0, 0.0).astype(jnp.float32)
                plsc.addupdate_scatter(acc_v, [10 * _ACCW + seg16], fgf)
            return carry

        lax.fori_loop(0, _NVREG, body, 0)

    _issue(_m_desc(0, 0))
    _issue(_in_desc(0, 0))
    _issue(_in_desc(0, 1))

    for jt in range(_TPT):
        pb = jt % 2
        _wait(_m_desc(jt, pb))
        if jt + 1 < _TPT:
            _issue(_m_desc(jt + 1, 1 - pb))
        _issue(_in_desc(jt, 4))
        _wait(_in_desc(jt, 0))
        _wait(_in_desc(jt, 1))
        if jt > 0:
            _wait_out(jt - 1, 2)
            _wait_out(jt - 1, 3)
        _pass(jt, pb, 0)
        _issue(_in_desc(jt, 2))
        _issue(_in_desc(jt, 3))
        _issue_out(jt, 0)
        _wait(_in_desc(jt, 2))
        _wait_out(jt, 0)
        _pass(jt, pb, 1)
        if jt + 1 < _TPT:
            _issue(_in_desc(jt + 1, 1))
        _issue_out(jt, 1)
        _wait(_in_desc(jt, 3))
        _wait_out(jt, 1)
        _pass(jt, pb, 2)
        if jt + 1 < _TPT:
            _issue(_in_desc(jt + 1, 0))
        _issue_out(jt, 2)
        _wait(_in_desc(jt, 4))
        _pass(jt, pb, 3)
        _issue_out(jt, 3)

    _wait_out(_TPT - 1, 2)
    _wait_out(_TPT - 1, 3)
    off = pl.multiple_of(wid * _ACCN, 128)
    pltpu.sync_copy(acc_v, part_hbm.at[pl.ds(off, _ACCN)])


_sc_main = functools.partial(
    pl.kernel,
    out_type=(
        jax.ShapeDtypeStruct((_B, 4, _H, _W), jnp.float32),
        jax.ShapeDtypeStruct((_B, 3, _H, _W), jnp.float32),
        jax.ShapeDtypeStruct((_B, 2, _H, _W), jnp.float32),
        jax.ShapeDtypeStruct((_B, _H, _W), jnp.float32),
        jax.ShapeDtypeStruct((_NW * _ACCN,), jnp.float32),
    ),
    mesh=_sc_mesh,
    compiler_params=pltpu.CompilerParams(needs_layout_passes=False),
    scratch_types=[
        pltpu.VMEM((2, 8, _W), jnp.int32),
        pltpu.VMEM((24, 8, _W), jnp.float32),
        pltpu.VMEM((24, 8, _W), jnp.float32),
        pltpu.VMEM((4, 8, _W), jnp.float32),
        pltpu.VMEM((_ACCN,), jnp.float32),
        pltpu.SemaphoreType.DMA,
        pltpu.SemaphoreType.DMA,
        pltpu.SemaphoreType.DMA,
        pltpu.SemaphoreType.DMA,
        pltpu.SemaphoreType.DMA,
        pltpu.SemaphoreType.DMA,
        pltpu.SemaphoreType.DMA,
        pltpu.SemaphoreType.DMA,
    ],
)(_sc_body)


def _fin_body(p_ref, aq_ref, as_ref, axy_ref, az_ref, rt_ref, cnt_ref):
    w = _NW * 16
    cols = []
    for v in range(_NVAL):
        cols.append(jnp.sum(p_ref[:, v * w:(v + 1) * w], axis=1, keepdims=True))
    counts = cols[10]
    denom = jnp.maximum(counts, 1.0)
    q0, q1, q2, q3 = (cols[0] / denom, cols[1] / denom,
                      cols[2] / denom, cols[3] / denom)
    s0, s1, s2 = cols[4] / denom, cols[5] / denom, cols[6] / denom
    ax, ay = cols[7] / denom, cols[8] / denom
    azm = cols[9] / denom

    nrm = jnp.maximum(jnp.sqrt(q0 * q0 + q1 * q1 + q2 * q2 + q3 * q3), 1e-8)
    qw, qx, qy, qz = q0 / nrm, q1 / nrm, q2 / nrm, q3 / nrm
    r00 = 1.0 - 2.0 * (qy * qy + qz * qz)
    r01 = 2.0 * (qx * qy - qz * qw)
    r02 = 2.0 * (qx * qz + qy * qw)
    r10 = 2.0 * (qx * qy + qz * qw)
    r11 = 1.0 - 2.0 * (qx * qx + qz * qz)
    r12 = 2.0 * (qy * qz - qx * qw)
    r20 = 2.0 * (qx * qz - qy * qw)
    r21 = 2.0 * (qy * qz + qx * qw)
    r22 = 1.0 - 2.0 * (qx * qx + qy * qy)

    zval = jnp.exp(azm)
    ki = _KINV
    t0 = zval * (ki[0, 0] * ax + ki[0, 1] * ay + ki[0, 2])
    t1 = zval * (ki[1, 0] * ax + ki[1, 1] * ay + ki[1, 2])
    t2 = zval * (ki[2, 0] * ax + ki[2, 1] * ay + ki[2, 2])

    zc = jnp.zeros_like(q0)
    oc = jnp.ones_like(q0)
    rt_ref[:] = jnp.concatenate(
        [r00, r01, r02, t0,
         r10, r11, r12, t1,
         r20, r21, r22, t2,
         zc, zc, zc, oc], axis=1)
    aq_ref[:] = jnp.concatenate([q0, q1, q2, q3], axis=1)
    as_ref[:] = jnp.concatenate([s0, s1, s2], axis=1)
    axy_ref[:] = jnp.concatenate([ax, ay], axis=1)
    az_ref[:] = azm
    cnt_ref[:] = counts


_fin = pl.pallas_call(
    _fin_body,
    out_shape=(
        jax.ShapeDtypeStruct((_NSEG, 4), jnp.float32),
        jax.ShapeDtypeStruct((_NSEG, 3), jnp.float32),
        jax.ShapeDtypeStruct((_NSEG, 2), jnp.float32),
        jax.ShapeDtypeStruct((_NSEG, 1), jnp.float32),
        jax.ShapeDtypeStruct((_NSEG, 16), jnp.float32),
        jax.ShapeDtypeStruct((_NSEG, 1), jnp.float32),
    ),
)


@jax.jit
def kernel(cat_mask, quaternion, scales, xy, z):
    gq, gs, gxy, gz, part = _sc_main(
        cat_mask.astype(jnp.int32), quaternion, scales, xy, z)
    p4 = part.reshape(_NW, _NVAL, _NSEG, 16).transpose(2, 1, 0, 3)
    aq, ascl, axy_o, az, rt16, cnt = _fin(p4.reshape(_NSEG, _NVAL * _NW * 16))
    RT = rt16.reshape(_NSEG, 4, 4)
    return aq, ascl, axy_o, az, RT, cnt, gq, gs, gxy, gz
